# baseline (speedup 1.0000x reference)
import functools

import jax
import jax.numpy as jnp
from jax import lax
from jax.experimental import pallas as pl
from jax.experimental.pallas import tpu as pltpu
from jax.experimental.pallas import tpu_sc as plsc

D_EMB = 16
EMBEDDING_SIZE = 1000
BUFFER = 5
SCALE = EMBEDDING_SIZE / (2 * BUFFER)
NAN_PADDING = EMBEDDING_SIZE
BN_EPS = 1e-5

B_ROWS = 16384
B_COLS = 100
LANES = 128
TC_CHUNKS = 8
TC_BLOCK_ROWS = B_ROWS // TC_CHUNKS

NW = 32
ROWS_PER_W = B_ROWS // NW
CHUNK_ROWS = 16
N_CHUNKS = ROWS_PER_W // CHUNK_ROWS


def _tc_idx_body(x_ref, g_ref, b_ref, idx_ref, s_ref):
    p = pl.program_id(0)
    c = pl.program_id(1)

    def valid_mask(x):
        lane = lax.broadcasted_iota(jnp.int32, x.shape, 1)
        return (lane < B_COLS) & ~(x != x)

    @pl.when((p == 0) & (c == 0))
    def _init():
        s_ref[0] = 0.0
        s_ref[1] = 0.0
        s_ref[2] = 0.0

    @pl.when(p == 0)
    def _acc_sum():
        x = x_ref[...]
        v = valid_mask(x)
        s_ref[0] += jnp.sum(jnp.where(v, x, 0.0))
        s_ref[2] += jnp.sum(jnp.where(v, 1.0, 0.0))

    @pl.when((p == 0) & (c == TC_CHUNKS - 1))
    def _fin_mean():
        s_ref[3] = s_ref[0] / s_ref[2]

    @pl.when(p == 1)
    def _acc_var():
        x = x_ref[...]
        v = valid_mask(x)
        d = (x - s_ref[3]) ** 2
        s_ref[1] += jnp.sum(jnp.where(v, d, 0.0))

    @pl.when((p == 1) & (c == TC_CHUNKS - 1))
    def _fin_var():
        var = s_ref[1] / s_ref[2]
        s_ref[4] = jnp.sqrt(var + BN_EPS)

    @pl.when(p == 2)
    def _emit_idx():
        x = x_ref[...]
        normalized = (x - s_ref[3]) / s_ref[4] * g_ref[0] + b_ref[0]
        t = (normalized + BUFFER) * SCALE
        t = jnp.clip(t, 0.0, float(NAN_PADDING - 1))
        ii = t.astype(jnp.int32)
        idx_ref[...] = jnp.where(x != x, NAN_PADDING, ii)


def _compute_idx(x128, gamma, beta):
    return pl.pallas_call(
        _tc_idx_body,
        grid=(3, TC_CHUNKS),
        in_specs=[
            pl.BlockSpec((TC_BLOCK_ROWS, LANES), lambda p, c: (c, 0)),
            pl.BlockSpec(memory_space=pltpu.SMEM),
            pl.BlockSpec(memory_space=pltpu.SMEM),
        ],
        out_specs=pl.BlockSpec((TC_BLOCK_ROWS, LANES), lambda p, c: (c, 0)),
        out_shape=jax.ShapeDtypeStruct((B_ROWS, LANES), jnp.int32),
        scratch_shapes=[pltpu.SMEM((8,), jnp.float32)],
    )(x128, gamma, beta)


_SC_MESH = plsc.VectorSubcoreMesh(core_axis_name="c", subcore_axis_name="s")


@functools.partial(
    pl.kernel,
    mesh=_SC_MESH,
    compiler_params=pltpu.CompilerParams(use_tc_tiling_on_sc=False),
    out_type=jax.ShapeDtypeStruct((B_ROWS, B_COLS, D_EMB), jnp.float32),
    scratch_types=[
        pltpu.VMEM((CHUNK_ROWS, LANES), jnp.int32),
        pltpu.VMEM((CHUNK_ROWS, LANES), jnp.int32),
        pltpu.VMEM((CHUNK_ROWS, LANES, D_EMB), jnp.float32),
        pltpu.VMEM((CHUNK_ROWS, LANES, D_EMB), jnp.float32),
        pltpu.SemaphoreType.DMA,
        pltpu.SemaphoreType.DMA,
        pltpu.SemaphoreType.DMA,
    ],
)
def _sc_gather(idx_hbm, table_hbm, out_hbm, ibuf0, ibuf1, obuf0, obuf1,
               sem_g, sem_w0, sem_w1):
    wid = lax.axis_index("s") * 2 + lax.axis_index("c")
    row0 = wid * ROWS_PER_W
    ibufs = (ibuf0, ibuf1)
    obufs = (obuf0, obuf1)
    sems_w = (sem_w0, sem_w1)

    def load_idx(c, b):
        pltpu.sync_copy(idx_hbm.at[pl.ds(row0 + c * CHUNK_ROWS, CHUNK_ROWS)],
                        ibufs[b])

    def gather_chunk(b):
        handles = [
            pltpu.async_copy(table_hbm.at[ibufs[b].at[j]],
                             obufs[b].at[j], sem_g)
            for j in range(CHUNK_ROWS)
        ]
        for h in handles:
            h.wait()

    def start_writeback(c, b):
        for j in range(CHUNK_ROWS):
            pltpu.async_copy(
                obufs[b].at[j, pl.ds(0, B_COLS)],
                out_hbm.at[row0 + c * CHUNK_ROWS + j],
                sems_w[b],
            )

    def wait_writeback(b):
        for j in range(CHUNK_ROWS):
            pltpu.make_async_copy(
                out_hbm.at[row0], obufs[b].at[j, pl.ds(0, B_COLS)], sems_w[b]
            ).wait()

    load_idx(0, 0)
    load_idx(1, 1)
    gather_chunk(0)
    start_writeback(0, 0)
    load_idx(2, 0)
    gather_chunk(1)
    start_writeback(1, 1)
    load_idx(3, 1)

    def pair_body(k, carry):
        for b in range(2):
            c = 2 * k + b
            wait_writeback(b)
            gather_chunk(b)
            start_writeback(c, b)
            load_idx(c + 2, b)
        return carry

    lax.fori_loop(1, N_CHUNKS // 2 - 1, pair_body, 0)

    for b in range(2):
        c = N_CHUNKS - 2 + b
        wait_writeback(b)
        gather_chunk(b)
        start_writeback(c, b)
    wait_writeback(0)
    wait_writeback(1)


def kernel(input_tensor, gamma, beta, emb_table):
    x128 = jnp.pad(input_tensor, ((0, 0), (0, LANES - B_COLS)))
    idx = _compute_idx(x128, gamma, beta)
    return _sc_gather(idx, emb_table)


# speedup vs baseline: 2.2029x; 2.2029x over previous
import functools

import jax
import jax.numpy as jnp
from jax import lax
from jax.experimental import pallas as pl
from jax.experimental.pallas import tpu as pltpu
from jax.experimental.pallas import tpu_sc as plsc

D_EMB = 16
EMBEDDING_SIZE = 1000
BUFFER = 5
SCALE = EMBEDDING_SIZE / (2 * BUFFER)
NAN_PADDING = EMBEDDING_SIZE
BN_EPS = 1e-5

B_ROWS = 16384
B_COLS = 100
LANES = 128
TC_CHUNKS = 8
TC_BLOCK_ROWS = B_ROWS // TC_CHUNKS

G_COLS = 104
NW = 32
ROWS_PER_W = B_ROWS // NW
CHUNK_ROWS = 16
N_CHUNKS = ROWS_PER_W // CHUNK_ROWS


def _tc_idx_body(x_ref, g_ref, b_ref, idx_ref, s_ref):
    p = pl.program_id(0)
    c = pl.program_id(1)

    def valid_mask(x):
        lane = lax.broadcasted_iota(jnp.int32, x.shape, 1)
        return (lane < B_COLS) & ~(x != x)

    @pl.when((p == 0) & (c == 0))
    def _init():
        s_ref[0] = 0.0
        s_ref[1] = 0.0
        s_ref[2] = 0.0

    @pl.when(p == 0)
    def _acc_sum():
        x = x_ref[...]
        v = valid_mask(x)
        s_ref[0] += jnp.sum(jnp.where(v, x, 0.0))
        s_ref[2] += jnp.sum(jnp.where(v, 1.0, 0.0))

    @pl.when((p == 0) & (c == TC_CHUNKS - 1))
    def _fin_mean():
        s_ref[3] = s_ref[0] / s_ref[2]

    @pl.when(p == 1)
    def _acc_var():
        x = x_ref[...]
        v = valid_mask(x)
        d = (x - s_ref[3]) ** 2
        s_ref[1] += jnp.sum(jnp.where(v, d, 0.0))

    @pl.when((p == 1) & (c == TC_CHUNKS - 1))
    def _fin_var():
        var = s_ref[1] / s_ref[2]
        s_ref[4] = jnp.sqrt(var + BN_EPS)

    @pl.when(p == 2)
    def _emit_idx():
        x = x_ref[...]
        normalized = (x - s_ref[3]) / s_ref[4] * g_ref[0] + b_ref[0]
        t = (normalized + BUFFER) * SCALE
        t = jnp.clip(t, 0.0, float(NAN_PADDING - 1))
        ii = t.astype(jnp.int32)
        idx_ref[...] = jnp.where(x != x, NAN_PADDING, ii)


def _compute_idx(x128, gamma, beta):
    return pl.pallas_call(
        _tc_idx_body,
        grid=(3, TC_CHUNKS),
        in_specs=[
            pl.BlockSpec((TC_BLOCK_ROWS, LANES), lambda p, c: (c, 0)),
            pl.BlockSpec(memory_space=pltpu.SMEM),
            pl.BlockSpec(memory_space=pltpu.SMEM),
        ],
        out_specs=pl.BlockSpec((TC_BLOCK_ROWS, LANES), lambda p, c: (c, 0)),
        out_shape=jax.ShapeDtypeStruct((B_ROWS, LANES), jnp.int32),
        scratch_shapes=[pltpu.SMEM((8,), jnp.float32)],
    )(x128, gamma, beta)


_SC_MESH = plsc.VectorSubcoreMesh(core_axis_name="c", subcore_axis_name="s")


@functools.partial(
    pl.kernel,
    mesh=_SC_MESH,
    compiler_params=pltpu.CompilerParams(use_tc_tiling_on_sc=False),
    out_type=jax.ShapeDtypeStruct((B_ROWS, B_COLS, D_EMB), jnp.float32),
    scratch_types=[
        pltpu.VMEM((CHUNK_ROWS, G_COLS), jnp.int32),
        pltpu.VMEM((CHUNK_ROWS, G_COLS), jnp.int32),
        pltpu.VMEM((CHUNK_ROWS, G_COLS, D_EMB), jnp.float32),
        pltpu.VMEM((CHUNK_ROWS, G_COLS, D_EMB), jnp.float32),
        pltpu.SemaphoreType.DMA,
        pltpu.SemaphoreType.DMA,
        pltpu.SemaphoreType.DMA,
    ],
)
def _sc_gather(idx_hbm, table_hbm, out_hbm, ibuf0, ibuf1,
               obuf0, obuf1, sem_g, sem_w0, sem_w1):
    wid = lax.axis_index("s") * 2 + lax.axis_index("c")
    row0 = wid * ROWS_PER_W
    ibufs = (ibuf0, ibuf1)
    obufs = (obuf0, obuf1)
    sems_w = (sem_w0, sem_w1)

    def load_idx(c, b):
        pltpu.sync_copy(
            idx_hbm.at[pl.ds(row0 + c * CHUNK_ROWS, CHUNK_ROWS),
                       pl.ds(0, G_COLS)],
            ibufs[b])

    def start_writeback(c, b):
        for j in range(CHUNK_ROWS):
            pltpu.async_copy(
                obufs[b].at[j, pl.ds(0, B_COLS)],
                out_hbm.at[row0 + c * CHUNK_ROWS + j],
                sems_w[b],
            )

    def wait_writeback(b):
        for j in range(CHUNK_ROWS):
            pltpu.make_async_copy(
                out_hbm.at[row0], obufs[b].at[j, pl.ds(0, B_COLS)], sems_w[b]
            ).wait()

    def run_chunk(c, b, prefetch):
        handles = [
            pltpu.async_copy(table_hbm.at[ibufs[b].at[j]],
                             obufs[b].at[j], sem_g)
            for j in range(CHUNK_ROWS)
        ]
        for h in handles:
            h.wait()
        start_writeback(c, b)
        if prefetch:
            load_idx(c + 2, b)

    load_idx(0, 0)
    load_idx(1, 1)
    run_chunk(0, 0, True)
    run_chunk(1, 1, True)

    def pair_body(k, carry):
        for b in range(2):
            c = 2 * k + b
            wait_writeback(b)
            run_chunk(c, b, True)
        return carry

    lax.fori_loop(1, N_CHUNKS // 2 - 1, pair_body, 0)

    for b in range(2):
        c = N_CHUNKS - 2 + b
        wait_writeback(b)
        run_chunk(c, b, False)
    wait_writeback(0)
    wait_writeback(1)


def kernel(input_tensor, gamma, beta, emb_table):
    x128 = jnp.pad(input_tensor, ((0, 0), (0, LANES - B_COLS)))
    idx = _compute_idx(x128, gamma, beta)
    return _sc_gather(idx, emb_table)


# speedup vs baseline: 2.2044x; 1.0007x over previous
import functools

import jax
import jax.numpy as jnp
from jax import lax
from jax.experimental import pallas as pl
from jax.experimental.pallas import tpu as pltpu
from jax.experimental.pallas import tpu_sc as plsc

D_EMB = 16
EMBEDDING_SIZE = 1000
BUFFER = 5
SCALE = EMBEDDING_SIZE / (2 * BUFFER)
NAN_PADDING = EMBEDDING_SIZE
BN_EPS = 1e-5

B_ROWS = 16384
B_COLS = 100
LANES = 128
TC_CHUNKS = 8
TC_BLOCK_ROWS = B_ROWS // TC_CHUNKS

G_COLS = 104
NW = 32
ROWS_PER_W = B_ROWS // NW
CHUNK_ROWS = 16
N_CHUNKS = ROWS_PER_W // CHUNK_ROWS


def _tc_idx_body(x_ref, g_ref, b_ref, idx_ref, s_ref):
    p = pl.program_id(0)
    c = pl.program_id(1)

    def valid_mask(x):
        lane = lax.broadcasted_iota(jnp.int32, x.shape, 1)
        return (lane < B_COLS) & ~(x != x)

    @pl.when((p == 0) & (c == 0))
    def _init():
        s_ref[0] = 0.0
        s_ref[1] = 0.0
        s_ref[2] = 0.0

    @pl.when(p == 0)
    def _acc_sum():
        x = x_ref[...]
        v = valid_mask(x)
        s_ref[0] += jnp.sum(jnp.where(v, x, 0.0))
        s_ref[2] += jnp.sum(jnp.where(v, 1.0, 0.0))

    @pl.when((p == 0) & (c == TC_CHUNKS - 1))
    def _fin_mean():
        s_ref[3] = s_ref[0] / s_ref[2]

    @pl.when(p == 1)
    def _acc_var():
        x = x_ref[...]
        v = valid_mask(x)
        d = (x - s_ref[3]) ** 2
        s_ref[1] += jnp.sum(jnp.where(v, d, 0.0))

    @pl.when((p == 1) & (c == TC_CHUNKS - 1))
    def _fin_var():
        var = s_ref[1] / s_ref[2]
        s_ref[4] = jnp.sqrt(var + BN_EPS)

    @pl.when(p == 2)
    def _emit_idx():
        x = x_ref[...]
        normalized = (x - s_ref[3]) / s_ref[4] * g_ref[0] + b_ref[0]
        t = (normalized + BUFFER) * SCALE
        t = jnp.clip(t, 0.0, float(NAN_PADDING - 1))
        ii = t.astype(jnp.int32)
        idx_ref[...] = jnp.where(x != x, NAN_PADDING, ii)


def _compute_idx(x128, gamma, beta):
    return pl.pallas_call(
        _tc_idx_body,
        grid=(3, TC_CHUNKS),
        in_specs=[
            pl.BlockSpec((TC_BLOCK_ROWS, LANES), lambda p, c: (c, 0)),
            pl.BlockSpec(memory_space=pltpu.SMEM),
            pl.BlockSpec(memory_space=pltpu.SMEM),
        ],
        out_specs=pl.BlockSpec((TC_BLOCK_ROWS, LANES), lambda p, c: (c, 0)),
        out_shape=jax.ShapeDtypeStruct((B_ROWS, LANES), jnp.int32),
        scratch_shapes=[pltpu.SMEM((8,), jnp.float32)],
    )(x128, gamma, beta)


_SC_MESH = plsc.VectorSubcoreMesh(core_axis_name="c", subcore_axis_name="s")


@functools.partial(
    pl.kernel,
    mesh=_SC_MESH,
    compiler_params=pltpu.CompilerParams(use_tc_tiling_on_sc=False),
    out_type=jax.ShapeDtypeStruct((B_ROWS, B_COLS, D_EMB), jnp.float32),
    scratch_types=[
        pltpu.VMEM((CHUNK_ROWS, G_COLS), jnp.int32),
        pltpu.VMEM((CHUNK_ROWS, G_COLS), jnp.int32),
        pltpu.VMEM((CHUNK_ROWS, G_COLS, D_EMB), jnp.float32),
        pltpu.VMEM((CHUNK_ROWS, G_COLS, D_EMB), jnp.float32),
        pltpu.SemaphoreType.DMA,
        pltpu.SemaphoreType.DMA,
        pltpu.SemaphoreType.DMA,
    ],
)
def _sc_gather(idx_hbm, table_hbm, out_hbm, ibuf0, ibuf1,
               obuf0, obuf1, sem_g, sem_w0, sem_w1):
    wid = lax.axis_index("s") * 2 + lax.axis_index("c")
    row0 = wid * ROWS_PER_W
    ibufs = (ibuf0, ibuf1)
    obufs = (obuf0, obuf1)
    sems_w = (sem_w0, sem_w1)

    def load_idx(c, b):
        pltpu.sync_copy(
            idx_hbm.at[pl.ds(row0 + c * CHUNK_ROWS, CHUNK_ROWS),
                       pl.ds(0, G_COLS)],
            ibufs[b])

    def start_writeback(c, b):
        pltpu.async_copy(
            obufs[b].at[:, pl.ds(0, B_COLS)],
            out_hbm.at[pl.ds(row0 + c * CHUNK_ROWS, CHUNK_ROWS)],
            sems_w[b],
        )

    def wait_writeback(b):
        pltpu.make_async_copy(
            out_hbm.at[pl.ds(row0, CHUNK_ROWS)],
            obufs[b].at[:, pl.ds(0, B_COLS)],
            sems_w[b],
        ).wait()

    def run_chunk(c, b, prefetch):
        handles = [
            pltpu.async_copy(table_hbm.at[ibufs[b].at[j]],
                             obufs[b].at[j], sem_g)
            for j in range(CHUNK_ROWS)
        ]
        for h in handles:
            h.wait()
        start_writeback(c, b)
        if prefetch:
            load_idx(c + 2, b)

    load_idx(0, 0)
    load_idx(1, 1)
    run_chunk(0, 0, True)
    run_chunk(1, 1, True)

    def pair_body(k, carry):
        for b in range(2):
            c = 2 * k + b
            wait_writeback(b)
            run_chunk(c, b, True)
        return carry

    lax.fori_loop(1, N_CHUNKS // 2 - 1, pair_body, 0)

    for b in range(2):
        c = N_CHUNKS - 2 + b
        wait_writeback(b)
        run_chunk(c, b, False)
    wait_writeback(0)
    wait_writeback(1)


def kernel(input_tensor, gamma, beta, emb_table):
    x128 = jnp.pad(input_tensor, ((0, 0), (0, LANES - B_COLS)))
    idx = _compute_idx(x128, gamma, beta)
    return _sc_gather(idx, emb_table)


# speedup vs baseline: 2.7177x; 1.2328x over previous
import functools

import jax
import jax.numpy as jnp
from jax import lax
from jax.experimental import pallas as pl
from jax.experimental.pallas import tpu as pltpu
from jax.experimental.pallas import tpu_sc as plsc

D_EMB = 16
EMBEDDING_SIZE = 1000
BUFFER = 5
SCALE = EMBEDDING_SIZE / (2 * BUFFER)
NAN_PADDING = EMBEDDING_SIZE
BN_EPS = 1e-5

B_ROWS = 16384
B_COLS = 100
TC_CHUNKS = 8
TC_BLOCK_ROWS = B_ROWS // TC_CHUNKS

NW = 32
ROWS_PER_W = B_ROWS // NW
CHUNK_ROWS = 32
N_CHUNKS = ROWS_PER_W // CHUNK_ROWS


def _tc_idx_body(x_ref, g_ref, b_ref, idx_ref, s_ref):
    p = pl.program_id(0)
    c = pl.program_id(1)

    @pl.when((p == 0) & (c == 0))
    def _init():
        s_ref[0] = 0.0
        s_ref[1] = 0.0
        s_ref[2] = 0.0

    @pl.when(p == 0)
    def _acc_sum():
        x = x_ref[...]
        nan = x != x
        xv = jnp.where(nan, 0.0, x)
        s_ref[0] += jnp.sum(xv)
        s_ref[2] += jnp.sum(jnp.where(nan, 0.0, 1.0))

    @pl.when((p == 0) & (c == TC_CHUNKS - 1))
    def _fin_mean():
        s_ref[3] = s_ref[0] / s_ref[2]

    @pl.when(p == 1)
    def _acc_var():
        x = x_ref[...]
        nan = x != x
        d = (x - s_ref[3]) ** 2
        s_ref[1] += jnp.sum(jnp.where(nan, 0.0, d))

    @pl.when((p == 1) & (c == TC_CHUNKS - 1))
    def _fin_var():
        var = s_ref[1] / s_ref[2]
        s_ref[4] = jnp.sqrt(var + BN_EPS)

    @pl.when(p == 2)
    def _emit_idx():
        x = x_ref[...]
        normalized = (x - s_ref[3]) / s_ref[4] * g_ref[0] + b_ref[0]
        t = (normalized + BUFFER) * SCALE
        t = jnp.clip(t, 0.0, float(NAN_PADDING - 1))
        ii = t.astype(jnp.int32)
        idx_ref[...] = jnp.where(x != x, NAN_PADDING, ii)


def _compute_idx(x, gamma, beta):
    return pl.pallas_call(
        _tc_idx_body,
        grid=(3, TC_CHUNKS),
        in_specs=[
            pl.BlockSpec((TC_BLOCK_ROWS, B_COLS), lambda p, c: (c, 0)),
            pl.BlockSpec(memory_space=pltpu.SMEM),
            pl.BlockSpec(memory_space=pltpu.SMEM),
        ],
        out_specs=pl.BlockSpec((TC_BLOCK_ROWS, B_COLS), lambda p, c: (c, 0)),
        out_shape=jax.ShapeDtypeStruct((B_ROWS, B_COLS), jnp.int32),
        scratch_shapes=[pltpu.SMEM((8,), jnp.float32)],
    )(x, gamma, beta)


_SC_MESH = plsc.VectorSubcoreMesh(core_axis_name="c", subcore_axis_name="s")


@functools.partial(
    pl.kernel,
    mesh=_SC_MESH,
    compiler_params=pltpu.CompilerParams(use_tc_tiling_on_sc=False),
    out_type=jax.ShapeDtypeStruct((B_ROWS, B_COLS, D_EMB), jnp.float32),
    scratch_types=[
        pltpu.VMEM((CHUNK_ROWS, B_COLS), jnp.int32),
        pltpu.VMEM((CHUNK_ROWS, B_COLS), jnp.int32),
        pltpu.VMEM((CHUNK_ROWS, B_COLS, D_EMB), jnp.float32),
        pltpu.VMEM((CHUNK_ROWS, B_COLS, D_EMB), jnp.float32),
        pltpu.SemaphoreType.DMA,
        pltpu.SemaphoreType.DMA,
        pltpu.SemaphoreType.DMA,
    ],
)
def _sc_gather(idx_hbm, table_hbm, out_hbm, ibuf0, ibuf1, obuf0, obuf1,
               sem_g, sem_w0, sem_w1):
    wid = lax.axis_index("s") * 2 + lax.axis_index("c")
    row0 = wid * ROWS_PER_W
    ibufs = (ibuf0, ibuf1)
    obufs = (obuf0, obuf1)
    sems_w = (sem_w0, sem_w1)

    def load_idx(c, b):
        pltpu.sync_copy(idx_hbm.at[pl.ds(row0 + c * CHUNK_ROWS, CHUNK_ROWS)],
                        ibufs[b])

    def gather_chunk(b):
        handles = [
            pltpu.async_copy(table_hbm.at[ibufs[b].at[j]],
                             obufs[b].at[j], sem_g)
            for j in range(CHUNK_ROWS)
        ]
        for h in handles:
            h.wait()

    def start_writeback(c, b):
        pltpu.async_copy(
            obufs[b],
            out_hbm.at[pl.ds(row0 + c * CHUNK_ROWS, CHUNK_ROWS)],
            sems_w[b],
        )

    def wait_writeback(b):
        pltpu.make_async_copy(
            out_hbm.at[pl.ds(row0, CHUNK_ROWS)], obufs[b], sems_w[b]
        ).wait()

    load_idx(0, 0)
    load_idx(1, 1)
    gather_chunk(0)
    start_writeback(0, 0)
    load_idx(2, 0)
    gather_chunk(1)
    start_writeback(1, 1)
    load_idx(3, 1)

    def pair_body(k, carry):
        for b in range(2):
            c = 2 * k + b
            wait_writeback(b)
            gather_chunk(b)
            start_writeback(c, b)
            load_idx(c + 2, b)
        return carry

    lax.fori_loop(1, N_CHUNKS // 2 - 1, pair_body, 0)

    for b in range(2):
        c = N_CHUNKS - 2 + b
        wait_writeback(b)
        gather_chunk(b)
        start_writeback(c, b)
    wait_writeback(0)
    wait_writeback(1)


def kernel(input_tensor, gamma, beta, emb_table):
    idx = _compute_idx(input_tensor, gamma, beta)
    return _sc_gather(idx, emb_table)
